# baseline (speedup 1.0000x reference)
import jax
import jax.numpy as jnp
from jax import lax
from jax.experimental import pallas as pl
from jax.experimental.pallas import tpu as pltpu
from jax.experimental.pallas import tpu_sc as plsc

B = 4
D = 900000
S = 100000
NP = 200000
N_ENT = NP * 9

NCORES = 2
NTILES = 16
PT = 12544
PT15 = NP - 15 * PT
GP = 128
G_ENT = GP * 9
GW = G_ENT
NG = PT // GP
NG15 = PT15 // GP
TPAIR = PT15 - NG15 * GP
T_ENT = TPAIR * 9
UCH = 1568
UCH15 = PT15 - 7 * UCH

HALF = 1800000
TRASH_BASE = 1800000
TRASH_MASK = 32767
CW = 1200
C_WORDS = 1833600
NZCH = C_WORDS // CW
NWCH = HALF // CW

_GDN = lax.GatherDimensionNumbers(
    offset_dims=(), collapsed_slice_dims=(0,), start_index_map=(0,))


def _vperm(x, idx):
    return lax.gather(x, idx[:, None], _GDN, (1,),
                      mode=lax.GatherScatterMode.PROMISE_IN_BOUNDS)


def _sc_body(gi1, gi2, ulin, us, h, out, fnhb,
             ib0, ib1, xb0, xb1, vb0, vb1, fc0, fc1, ulbc, uvbc,
             cbA, cbB, hbA, hbB, C,
             semg, semI0, semI1, semF0, semF1, semS0, semS1,
             semC0, semC1, semH0, semH1, semO0, semO1):
    sc = lax.axis_index("c")
    s = lax.axis_index("s")
    base = sc * HALF
    last = s == NTILES - 1

    lane = lax.iota(jnp.int32, 16)
    zero16 = lane.astype(jnp.float32) * 0.0
    def _z1(i, c):
        cbA[pl.ds(i * 16, 16)] = zero16
        return c
    lax.fori_loop(0, CW // 16, _z1, 0)

    def _z2(i, c):
        k = i * 32 + s * 2

        @pl.when(k < NZCH)
        def _():
            pltpu.sync_copy(cbA, C.at[pl.ds(k * CW, CW)])

        @pl.when(k + 1 < NZCH)
        def _():
            pltpu.sync_copy(cbA, C.at[pl.ds((k + 1) * CW, CW)])
        return c
    lax.fori_loop(0, (NZCH + 31) // 32, _z2, 0)

    pbase = s * PT
    fbase = sc * NP + pbase

    def _flags(n16):
        def _fk(r, c):
            u = uvbc[pl.ds(r * 16, 16)]
            uvbc[pl.ds(r * 16, 16)] = jnp.where(
                u < 0.0, jnp.float32(-2.0), jnp.float32(0.0))
            return c
        lax.fori_loop(0, n16, _fk, 0)

    for i in range(7):
        ub = pbase + i * UCH
        pltpu.sync_copy(ulin.at[pl.ds(ub, UCH)], ulbc)
        pltpu.async_copy(us.at[ulbc], uvbc, semg).wait()
        _flags(UCH // 16)
        pltpu.sync_copy(uvbc, fnhb.at[pl.ds(sc * NP + ub, UCH)])

    @pl.when(jnp.logical_not(last))
    def _():
        ub = pbase + 7 * UCH
        pltpu.sync_copy(ulin.at[pl.ds(ub, UCH)], ulbc)
        pltpu.async_copy(us.at[ulbc], uvbc, semg).wait()
        _flags(UCH // 16)
        pltpu.sync_copy(uvbc, fnhb.at[pl.ds(sc * NP + ub, UCH)])

    @pl.when(last)
    def _():
        ub = pbase + 7 * UCH
        pltpu.sync_copy(ulin.at[pl.ds(ub, UCH15)], ulbc.at[pl.ds(0, UCH15)])
        pltpu.async_copy(us.at[ulbc.at[pl.ds(0, UCH15)]],
                         uvbc.at[pl.ds(0, UCH15)], semg).wait()
        _flags(UCH15 // 16)
        pltpu.sync_copy(uvbc.at[pl.ds(0, UCH15)],
                        fnhb.at[pl.ds(sc * NP + ub, UCH15)])

    plsc.subcore_barrier()

    permj, flipj = [], []
    for j in range(9):
        t = lane + 16 * j
        p = lax.shift_right_logical(t * 57, 9)
        e = t - p * 9
        permj.append(p)
        flipj.append(jnp.where((e == 2) | (e == 5) | (e == 6) | (e == 7),
                               jnp.float32(1.0), jnp.float32(0.0)))

    wbase = pbase * 9
    ng = jnp.where(last, NG15, NG)

    def _compute_chunk(ib, xb, vb, fc, cc):
        f16 = fc[pl.ds(cc * 16, 16)]
        tb = cc * 144
        for j in range(9):
            t = tb + j * 16
            lv = ib[pl.ds(t, 16)]
            off = lv - base
            inr = (off >= 0) & (off < HALF)
            tr = TRASH_BASE + lax.bitwise_and(lv, TRASH_MASK)
            xb[pl.ds(t, 16)] = jnp.where(inr, off, tr)
            v = _vperm(f16, permj[j]) * flipj[j] + 1.0
            vb[pl.ds(t, 16)] = v

    def _scatter_array(gi_hbm):
        def _issue_in(g, ib, fc, semI, semF):
            pltpu.async_copy(gi_hbm.at[pl.ds(wbase + g * GW, GW)], ib, semI)
            pltpu.async_copy(fnhb.at[pl.ds(fbase + g * GP, GP)], fc, semF)

        def _do_group(g, ib, xb, vb, fc, semI, semF, semS):
            pltpu.make_async_copy(gi_hbm.at[pl.ds(0, GW)], ib, semI).wait()
            pltpu.make_async_copy(fnhb.at[pl.ds(0, GP)], fc, semF).wait()

            @pl.when(g >= 2)
            def _():
                pltpu.make_async_copy(vb, C.at[xb], semS).wait()

            def _chunk(cc, c2):
                _compute_chunk(ib, xb, vb, fc, cc)
                return c2
            lax.fori_loop(0, GP // 16, _chunk, 0)
            pltpu.async_copy(vb, C.at[xb], semS, add=True)

            @pl.when(g + 2 < ng)
            def _():
                _issue_in(g + 2, ib, fc, semI, semF)

        _issue_in(0, ib0, fc0, semI0, semF0)
        _issue_in(1, ib1, fc1, semI1, semF1)

        def _gg(gg, c):
            g = gg * 2
            _do_group(g, ib0, xb0, vb0, fc0, semI0, semF0, semS0)
            _do_group(g + 1, ib1, xb1, vb1, fc1, semI1, semF1, semS1)
            return c
        lax.fori_loop(0, jnp.where(last, NG15 // 2, NG // 2), _gg, 0)

        pltpu.make_async_copy(vb0, C.at[xb0], semS0).wait()
        pltpu.make_async_copy(vb1, C.at[xb1], semS1).wait()

        @pl.when(last)
        def _():
            pltpu.sync_copy(gi_hbm.at[pl.ds(wbase + NG15 * GW, T_ENT)],
                            ib0.at[pl.ds(0, T_ENT)])
            pltpu.sync_copy(fnhb.at[pl.ds(fbase + NG15 * GP, TPAIR)],
                            fc0.at[pl.ds(0, TPAIR)])

            def _tchunk(cc, c2):
                _compute_chunk(ib0, xb0, vb0, fc0, cc)
                return c2
            lax.fori_loop(0, TPAIR // 16, _tchunk, 0)

            def _tz(q, c2):
                vb0[pl.ds(T_ENT + q * 16, 16)] = zero16
                return c2
            lax.fori_loop(0, (G_ENT - T_ENT) // 16, _tz, 0)
            pltpu.async_copy(vb0, C.at[xb0], semS0, add=True)
            pltpu.make_async_copy(vb0, C.at[xb0], semS0).wait()

    _scatter_array(gi1)
    _scatter_array(gi2)

    plsc.subcore_barrier()

    def _mul(cb, hb):
        def _m(q, c2):
            q16 = q * 16
            cb[pl.ds(q16, 16)] = cb[pl.ds(q16, 16)] * hb[pl.ds(q16, 16)]
            return c2
        lax.fori_loop(0, CW // 16, _m, 0)

    def _wo(i, c):
        k0 = i * 32 + s * 2
        k1 = k0 + 1

        @pl.when(k0 < NWCH)
        def _():
            @pl.when(i > 0)
            def _():
                pltpu.make_async_copy(cbA, out.at[pl.ds(0, CW)], semO0).wait()
            pltpu.async_copy(C.at[pl.ds(k0 * CW, CW)], cbA, semC0)
            pltpu.async_copy(h.at[pl.ds(base + k0 * CW, CW)], hbA, semH0)

        @pl.when(k1 < NWCH)
        def _():
            @pl.when(i > 0)
            def _():
                pltpu.make_async_copy(cbB, out.at[pl.ds(0, CW)], semO1).wait()
            pltpu.async_copy(C.at[pl.ds(k1 * CW, CW)], cbB, semC1)
            pltpu.async_copy(h.at[pl.ds(base + k1 * CW, CW)], hbB, semH1)

        @pl.when(k0 < NWCH)
        def _():
            pltpu.make_async_copy(C.at[pl.ds(0, CW)], cbA, semC0).wait()
            pltpu.make_async_copy(h.at[pl.ds(0, CW)], hbA, semH0).wait()
            _mul(cbA, hbA)
            pltpu.async_copy(cbA, out.at[pl.ds(base + k0 * CW, CW)], semO0)

        @pl.when(k1 < NWCH)
        def _():
            pltpu.make_async_copy(C.at[pl.ds(0, CW)], cbB, semC1).wait()
            pltpu.make_async_copy(h.at[pl.ds(0, CW)], hbB, semH1).wait()
            _mul(cbB, hbB)
            pltpu.async_copy(cbB, out.at[pl.ds(base + k1 * CW, CW)], semO1)
        return c
    lax.fori_loop(0, (NWCH + 31) // 32, _wo, 0)

    pltpu.make_async_copy(cbA, out.at[pl.ds(0, CW)], semO0).wait()
    pltpu.make_async_copy(cbB, out.at[pl.ds(0, CW)], semO1).wait()


@jax.jit
def _sc_call(gi1, gi2, ulin, us_flat, h_flat):
    dma = pltpu.SemaphoreType.DMA
    return pl.kernel(
        _sc_body,
        out_type=(jax.ShapeDtypeStruct((B * D,), jnp.float32),
                  jax.ShapeDtypeStruct((NCORES * NP,), jnp.float32)),
        mesh=plsc.VectorSubcoreMesh(core_axis_name="c", subcore_axis_name="s",
                                    num_cores=NCORES, num_subcores=NTILES),
        scratch_types=[
            pltpu.VMEM((GW,), jnp.int32),
            pltpu.VMEM((GW,), jnp.int32),
            pltpu.VMEM((G_ENT,), jnp.int32),
            pltpu.VMEM((G_ENT,), jnp.int32),
            pltpu.VMEM((G_ENT,), jnp.float32),
            pltpu.VMEM((G_ENT,), jnp.float32),
            pltpu.VMEM((GP,), jnp.float32),
            pltpu.VMEM((GP,), jnp.float32),
            pltpu.VMEM((UCH,), jnp.int32),
            pltpu.VMEM((UCH,), jnp.float32),
            pltpu.VMEM((CW,), jnp.float32),
            pltpu.VMEM((CW,), jnp.float32),
            pltpu.VMEM((CW,), jnp.float32),
            pltpu.VMEM((CW,), jnp.float32),
            pltpu.VMEM_SHARED((C_WORDS,), jnp.float32),
            dma, dma, dma, dma, dma, dma, dma,
            dma, dma, dma, dma, dma, dma,
        ],
    )(gi1, gi2, ulin, us_flat, h_flat)


def kernel(H, u_s, gather_idx, gather_idx2, u_s_gather_idx):
    lin1 = (gather_idx[:, 0] * D + gather_idx[:, 1]).astype(jnp.int32)
    lin2 = (gather_idx2[:, 0] * D + gather_idx2[:, 1]).astype(jnp.int32)
    ul = (u_s_gather_idx[:, 0] * S + u_s_gather_idx[:, 1]).astype(jnp.int32)
    out, _ = _sc_call(lin1, lin2, ul, u_s.reshape(-1), H.reshape(-1))
    return out.reshape(B, D)


# speedup vs baseline: 1.0013x; 1.0013x over previous
import jax
import jax.numpy as jnp
from jax import lax
from jax.experimental import pallas as pl
from jax.experimental.pallas import tpu as pltpu
from jax.experimental.pallas import tpu_sc as plsc

B = 4
D = 900000
S = 100000
NP = 200000
N_ENT = NP * 9

NCORES = 2
NTILES = 16
PT = 12544
PT15 = NP - 15 * PT
GP = 128
G_ENT = GP * 9
GW = G_ENT
NG = PT // GP
NG15 = PT15 // GP
TPAIR = PT15 - NG15 * GP
T_ENT = TPAIR * 9
UCH = 1568
UCH15 = PT15 - 7 * UCH

HALF = 1800000
TRASH_BASE = 1800000
TRASH_MASK = 2047
CW = 1200
C_WORDS = 1802400
NZCH = C_WORDS // CW
NWCH = HALF // CW

_GDN = lax.GatherDimensionNumbers(
    offset_dims=(), collapsed_slice_dims=(0,), start_index_map=(0,))


def _vperm(x, idx):
    return lax.gather(x, idx[:, None], _GDN, (1,),
                      mode=lax.GatherScatterMode.PROMISE_IN_BOUNDS)


def _sc_body(gi1, gi2, ulin, us, h, out, fnhb,
             ib0, ib1, xb0, xb1, vb0, vb1, fc0, fc1, ulbc, uvbc,
             cbA, cbB, hbA, hbB, C,
             semg, semI0, semI1, semF0, semF1, semS0, semS1,
             semC0, semC1, semH0, semH1, semO0, semO1):
    sc = lax.axis_index("c")
    s = lax.axis_index("s")
    base = sc * HALF
    last = s == NTILES - 1

    lane = lax.iota(jnp.int32, 16)
    zero16 = lane.astype(jnp.float32) * 0.0
    def _z1(i, c):
        cbA[pl.ds(i * 16, 16)] = zero16
        return c
    lax.fori_loop(0, CW // 16, _z1, 0)

    def _z2(i, c):
        k = i * 32 + s * 2

        @pl.when(k < NZCH)
        def _():
            pltpu.sync_copy(cbA, C.at[pl.ds(k * CW, CW)])

        @pl.when(k + 1 < NZCH)
        def _():
            pltpu.sync_copy(cbA, C.at[pl.ds((k + 1) * CW, CW)])
        return c
    lax.fori_loop(0, (NZCH + 31) // 32, _z2, 0)

    pbase = s * PT
    fbase = sc * NP + pbase

    def _flags(n16):
        def _fk(r, c):
            u = uvbc[pl.ds(r * 16, 16)]
            uvbc[pl.ds(r * 16, 16)] = jnp.where(
                u < 0.0, jnp.float32(-2.0), jnp.float32(0.0))
            return c
        lax.fori_loop(0, n16, _fk, 0)

    for i in range(7):
        ub = pbase + i * UCH
        pltpu.sync_copy(ulin.at[pl.ds(ub, UCH)], ulbc)
        pltpu.async_copy(us.at[ulbc], uvbc, semg).wait()
        _flags(UCH // 16)
        pltpu.sync_copy(uvbc, fnhb.at[pl.ds(sc * NP + ub, UCH)])

    @pl.when(jnp.logical_not(last))
    def _():
        ub = pbase + 7 * UCH
        pltpu.sync_copy(ulin.at[pl.ds(ub, UCH)], ulbc)
        pltpu.async_copy(us.at[ulbc], uvbc, semg).wait()
        _flags(UCH // 16)
        pltpu.sync_copy(uvbc, fnhb.at[pl.ds(sc * NP + ub, UCH)])

    @pl.when(last)
    def _():
        ub = pbase + 7 * UCH
        pltpu.sync_copy(ulin.at[pl.ds(ub, UCH15)], ulbc.at[pl.ds(0, UCH15)])
        pltpu.async_copy(us.at[ulbc.at[pl.ds(0, UCH15)]],
                         uvbc.at[pl.ds(0, UCH15)], semg).wait()
        _flags(UCH15 // 16)
        pltpu.sync_copy(uvbc.at[pl.ds(0, UCH15)],
                        fnhb.at[pl.ds(sc * NP + ub, UCH15)])

    plsc.subcore_barrier()

    permj, flipj = [], []
    for j in range(9):
        t = lane + 16 * j
        p = lax.shift_right_logical(t * 57, 9)
        e = t - p * 9
        permj.append(p)
        flipj.append(jnp.where((e == 2) | (e == 5) | (e == 6) | (e == 7),
                               jnp.float32(1.0), jnp.float32(0.0)))

    wbase = pbase * 9
    ng = jnp.where(last, NG15, NG)

    def _compute_chunk(ib, xb, vb, fc, cc):
        f16 = fc[pl.ds(cc * 16, 16)]
        tb = cc * 144
        for j in range(9):
            t = tb + j * 16
            lv = ib[pl.ds(t, 16)]
            off = lv - base
            inr = (off >= 0) & (off < HALF)
            tr = TRASH_BASE + lax.bitwise_and(lv, TRASH_MASK)
            xb[pl.ds(t, 16)] = jnp.where(inr, off, tr)
            v = _vperm(f16, permj[j]) * flipj[j] + 1.0
            vb[pl.ds(t, 16)] = v

    def _scatter_array(gi_hbm):
        def _issue_in(g, ib, fc, semI, semF):
            pltpu.async_copy(gi_hbm.at[pl.ds(wbase + g * GW, GW)], ib, semI)
            pltpu.async_copy(fnhb.at[pl.ds(fbase + g * GP, GP)], fc, semF)

        def _do_group(g, ib, xb, vb, fc, semI, semF, semS):
            pltpu.make_async_copy(gi_hbm.at[pl.ds(0, GW)], ib, semI).wait()
            pltpu.make_async_copy(fnhb.at[pl.ds(0, GP)], fc, semF).wait()

            @pl.when(g >= 2)
            def _():
                pltpu.make_async_copy(vb, C.at[xb], semS).wait()

            def _chunk(cc, c2):
                _compute_chunk(ib, xb, vb, fc, cc)
                return c2
            lax.fori_loop(0, GP // 16, _chunk, 0)
            pltpu.async_copy(vb, C.at[xb], semS, add=True)

            @pl.when(g + 2 < ng)
            def _():
                _issue_in(g + 2, ib, fc, semI, semF)

        _issue_in(0, ib0, fc0, semI0, semF0)
        _issue_in(1, ib1, fc1, semI1, semF1)

        def _gg(gg, c):
            g = gg * 2
            _do_group(g, ib0, xb0, vb0, fc0, semI0, semF0, semS0)
            _do_group(g + 1, ib1, xb1, vb1, fc1, semI1, semF1, semS1)
            return c
        lax.fori_loop(0, jnp.where(last, NG15 // 2, NG // 2), _gg, 0)

        pltpu.make_async_copy(vb0, C.at[xb0], semS0).wait()
        pltpu.make_async_copy(vb1, C.at[xb1], semS1).wait()

        @pl.when(last)
        def _():
            pltpu.sync_copy(gi_hbm.at[pl.ds(wbase + NG15 * GW, T_ENT)],
                            ib0.at[pl.ds(0, T_ENT)])
            pltpu.sync_copy(fnhb.at[pl.ds(fbase + NG15 * GP, TPAIR)],
                            fc0.at[pl.ds(0, TPAIR)])

            def _tchunk(cc, c2):
                _compute_chunk(ib0, xb0, vb0, fc0, cc)
                return c2
            lax.fori_loop(0, TPAIR // 16, _tchunk, 0)

            def _tz(q, c2):
                vb0[pl.ds(T_ENT + q * 16, 16)] = zero16
                return c2
            lax.fori_loop(0, (G_ENT - T_ENT) // 16, _tz, 0)
            pltpu.async_copy(vb0, C.at[xb0], semS0, add=True)
            pltpu.make_async_copy(vb0, C.at[xb0], semS0).wait()

    _scatter_array(gi1)
    _scatter_array(gi2)

    plsc.subcore_barrier()

    def _mul(cb, hb):
        def _m(q, c2):
            q16 = q * 16
            cb[pl.ds(q16, 16)] = cb[pl.ds(q16, 16)] * hb[pl.ds(q16, 16)]
            return c2
        lax.fori_loop(0, CW // 16, _m, 0)

    def _wo(i, c):
        k0 = i * 32 + s * 2
        k1 = k0 + 1

        @pl.when(k0 < NWCH)
        def _():
            @pl.when(i > 0)
            def _():
                pltpu.make_async_copy(cbA, out.at[pl.ds(0, CW)], semO0).wait()
            pltpu.async_copy(C.at[pl.ds(k0 * CW, CW)], cbA, semC0)
            pltpu.async_copy(h.at[pl.ds(base + k0 * CW, CW)], hbA, semH0)

        @pl.when(k1 < NWCH)
        def _():
            @pl.when(i > 0)
            def _():
                pltpu.make_async_copy(cbB, out.at[pl.ds(0, CW)], semO1).wait()
            pltpu.async_copy(C.at[pl.ds(k1 * CW, CW)], cbB, semC1)
            pltpu.async_copy(h.at[pl.ds(base + k1 * CW, CW)], hbB, semH1)

        @pl.when(k0 < NWCH)
        def _():
            pltpu.make_async_copy(C.at[pl.ds(0, CW)], cbA, semC0).wait()
            pltpu.make_async_copy(h.at[pl.ds(0, CW)], hbA, semH0).wait()
            _mul(cbA, hbA)
            pltpu.async_copy(cbA, out.at[pl.ds(base + k0 * CW, CW)], semO0)

        @pl.when(k1 < NWCH)
        def _():
            pltpu.make_async_copy(C.at[pl.ds(0, CW)], cbB, semC1).wait()
            pltpu.make_async_copy(h.at[pl.ds(0, CW)], hbB, semH1).wait()
            _mul(cbB, hbB)
            pltpu.async_copy(cbB, out.at[pl.ds(base + k1 * CW, CW)], semO1)
        return c
    lax.fori_loop(0, (NWCH + 31) // 32, _wo, 0)

    pltpu.make_async_copy(cbA, out.at[pl.ds(0, CW)], semO0).wait()
    pltpu.make_async_copy(cbB, out.at[pl.ds(0, CW)], semO1).wait()


@jax.jit
def _sc_call(gi1, gi2, ulin, us_flat, h_flat):
    dma = pltpu.SemaphoreType.DMA
    return pl.kernel(
        _sc_body,
        out_type=(jax.ShapeDtypeStruct((B * D,), jnp.float32),
                  jax.ShapeDtypeStruct((NCORES * NP,), jnp.float32)),
        mesh=plsc.VectorSubcoreMesh(core_axis_name="c", subcore_axis_name="s",
                                    num_cores=NCORES, num_subcores=NTILES),
        scratch_types=[
            pltpu.VMEM((GW,), jnp.int32),
            pltpu.VMEM((GW,), jnp.int32),
            pltpu.VMEM((G_ENT,), jnp.int32),
            pltpu.VMEM((G_ENT,), jnp.int32),
            pltpu.VMEM((G_ENT,), jnp.float32),
            pltpu.VMEM((G_ENT,), jnp.float32),
            pltpu.VMEM((GP,), jnp.float32),
            pltpu.VMEM((GP,), jnp.float32),
            pltpu.VMEM((UCH,), jnp.int32),
            pltpu.VMEM((UCH,), jnp.float32),
            pltpu.VMEM((CW,), jnp.float32),
            pltpu.VMEM((CW,), jnp.float32),
            pltpu.VMEM((CW,), jnp.float32),
            pltpu.VMEM((CW,), jnp.float32),
            pltpu.VMEM_SHARED((C_WORDS,), jnp.float32),
            dma, dma, dma, dma, dma, dma, dma,
            dma, dma, dma, dma, dma, dma,
        ],
    )(gi1, gi2, ulin, us_flat, h_flat)


def kernel(H, u_s, gather_idx, gather_idx2, u_s_gather_idx):
    lin1 = (gather_idx[:, 0] * D + gather_idx[:, 1]).astype(jnp.int32)
    lin2 = (gather_idx2[:, 0] * D + gather_idx2[:, 1]).astype(jnp.int32)
    ul = (u_s_gather_idx[:, 0] * S + u_s_gather_idx[:, 1]).astype(jnp.int32)
    out, _ = _sc_call(lin1, lin2, ul, u_s.reshape(-1), H.reshape(-1))
    return out.reshape(B, D)
